# Initial kernel scaffold; baseline (speedup 1.0000x reference)
import functools

import jax
import jax.numpy as jnp
from jax import lax
from jax.experimental import pallas as pl
from jax.experimental.pallas import tpu as pltpu
from jax.experimental.pallas import tpu_sc as plsc

N_PAD = 10240
NC, NS = 2, 16
NW = NC * NS
EDGE_K = 128


def _mesh():
    return plsc.VectorSubcoreMesh(
        core_axis_name="c", subcore_axis_name="s", num_cores=NC, num_subcores=NS
    )


def _deg_kernel(dst, nchunks):
    rows_per_tile = N_PAD // NS

    @functools.partial(
        pl.kernel,
        out_type=jax.ShapeDtypeStruct((NC, N_PAD), jnp.float32),
        mesh=_mesh(),
        scratch_types=[
            pltpu.VMEM((nchunks, EDGE_K), jnp.int32),
            pltpu.VMEM((EDGE_K,), jnp.float32),
            pltpu.VMEM((rows_per_tile,), jnp.float32),
            pltpu.VMEM_SHARED((N_PAD,), jnp.float32),
        ],
    )
    def k(dst_hbm, out_hbm, idx_v, ones_v, zrow_v, acc):
        c = lax.axis_index("c")
        s = lax.axis_index("s")
        wid = c * NS + s

        pltpu.sync_copy(dst_hbm.at[wid], idx_v)

        def fill_ones(t, _):
            ones_v[pl.ds(t * 16, 16)] = jnp.ones((16,), jnp.float32)
            return 0

        lax.fori_loop(0, EDGE_K // 16, fill_ones, 0)

        def fill_zero(t, _):
            zrow_v[pl.ds(t * 16, 16)] = jnp.zeros((16,), jnp.float32)
            return 0

        lax.fori_loop(0, rows_per_tile // 16, fill_zero, 0)

        pltpu.sync_copy(zrow_v, acc.at[pl.ds(s * rows_per_tile, rows_per_tile)])
        plsc.subcore_barrier()

        def body(j, _):
            pltpu.sync_copy(ones_v, acc.at[idx_v.at[j]], add=True)
            return 0

        lax.fori_loop(0, nchunks, body, 0)
        plsc.subcore_barrier()

        pltpu.sync_copy(
            acc.at[pl.ds(s * rows_per_tile, rows_per_tile)],
            out_hbm.at[c, pl.ds(s * rows_per_tile, rows_per_tile)],
        )

    return k(dst)


def _agg_kernel(src, dst, h, d, nchunks):
    rows_per_tile = N_PAD // NS
    zchunks = rows_per_tile // EDGE_K

    @functools.partial(
        pl.kernel,
        out_type=jax.ShapeDtypeStruct((NC, N_PAD, d), jnp.float32),
        mesh=_mesh(),
        scratch_types=[
            pltpu.VMEM((nchunks, EDGE_K), jnp.int32),
            pltpu.VMEM((nchunks, EDGE_K), jnp.int32),
            pltpu.VMEM((2, EDGE_K, d), jnp.float32),
            pltpu.VMEM((EDGE_K, d), jnp.float32),
            pltpu.VMEM_SHARED((N_PAD, d), jnp.float32),
            pltpu.SemaphoreType.DMA,
            pltpu.SemaphoreType.DMA,
        ],
    )
    def k(src_hbm, dst_hbm, h_hbm, out_hbm, src_v, dst_v, rows_v, zero_v, acc,
          sem0, sem1):
        c = lax.axis_index("c")
        s = lax.axis_index("s")
        wid = c * NS + s

        pltpu.sync_copy(src_hbm.at[wid], src_v)
        pltpu.sync_copy(dst_hbm.at[wid], dst_v)

        nlanes = d // 16

        def zbody(t, _):
            zero_v[t // nlanes, pl.ds((t % nlanes) * 16, 16)] = jnp.zeros(
                (16,), jnp.float32
            )
            return 0

        lax.fori_loop(0, EDGE_K * nlanes, zbody, 0)
        for r in range(zchunks):
            pltpu.sync_copy(
                zero_v, acc.at[pl.ds(s * rows_per_tile + r * EDGE_K, EDGE_K)]
            )
        plsc.subcore_barrier()

        pltpu.async_copy(h_hbm.at[src_v.at[0]], rows_v.at[0], sem0)

        def body2(jj, _):
            j0 = jj * 2
            pltpu.make_async_copy(h_hbm.at[src_v.at[j0]], rows_v.at[0], sem0).wait()

            @pl.when(j0 + 1 < nchunks)
            def _():
                pltpu.async_copy(h_hbm.at[src_v.at[j0 + 1]], rows_v.at[1], sem1)

            pltpu.sync_copy(rows_v.at[0], acc.at[dst_v.at[j0]], add=True)

            @pl.when(j0 + 1 < nchunks)
            def _():
                pltpu.make_async_copy(
                    h_hbm.at[src_v.at[j0 + 1]], rows_v.at[1], sem1
                ).wait()

                @pl.when(j0 + 2 < nchunks)
                def _():
                    pltpu.async_copy(h_hbm.at[src_v.at[j0 + 2]], rows_v.at[0], sem0)

                pltpu.sync_copy(rows_v.at[1], acc.at[dst_v.at[j0 + 1]], add=True)

            return 0

        lax.fori_loop(0, (nchunks + 1) // 2, body2, 0)
        plsc.subcore_barrier()

        for r in range(zchunks):
            base = s * rows_per_tile + r * EDGE_K
            pltpu.sync_copy(
                acc.at[pl.ds(base, EDGE_K)], out_hbm.at[c, pl.ds(base, EDGE_K)]
            )

    return k(src, dst, h)


def _mm_scale(x, w, deg0, deg1):
    m, kdim = x.shape
    _, d = w.shape
    bm = 1024

    def body(x_ref, w_ref, d0_ref, d1_ref, o_ref):
        deg = d0_ref[...] + d1_ref[...] + 1.0
        dis = lax.rsqrt(deg)
        o_ref[...] = (
            jnp.dot(x_ref[...], w_ref[...], preferred_element_type=jnp.float32) * dis
        )

    return pl.pallas_call(
        body,
        grid=(m // bm,),
        in_specs=[
            pl.BlockSpec((bm, kdim), lambda i: (i, 0)),
            pl.BlockSpec((kdim, d), lambda i: (0, 0)),
            pl.BlockSpec((bm, 1), lambda i: (i, 0)),
            pl.BlockSpec((bm, 1), lambda i: (i, 0)),
        ],
        out_specs=pl.BlockSpec((bm, d), lambda i: (i, 0)),
        out_shape=jax.ShapeDtypeStruct((m, d), jnp.float32),
    )(x, w, deg0, deg1)


def _combine_relu_mm_scale(a0, a1, h1, deg0, deg1, b1, w2):
    m, d = h1.shape
    _, d2 = w2.shape
    bm = 1024

    def body(a0_ref, a1_ref, h1_ref, d0_ref, d1_ref, b_ref, w_ref, o_ref):
        deg = d0_ref[...] + d1_ref[...] + 1.0
        dis = lax.rsqrt(deg)
        z = jnp.maximum(
            (a0_ref[...] + a1_ref[...] + h1_ref[...]) * dis + b_ref[...], 0.0
        )
        o_ref[...] = (
            jnp.dot(z, w_ref[...], preferred_element_type=jnp.float32) * dis
        )

    return pl.pallas_call(
        body,
        grid=(m // bm,),
        in_specs=[
            pl.BlockSpec((bm, d), lambda i: (i, 0)),
            pl.BlockSpec((bm, d), lambda i: (i, 0)),
            pl.BlockSpec((bm, d), lambda i: (i, 0)),
            pl.BlockSpec((bm, 1), lambda i: (i, 0)),
            pl.BlockSpec((bm, 1), lambda i: (i, 0)),
            pl.BlockSpec((1, d), lambda i: (0, 0)),
            pl.BlockSpec((d, d2), lambda i: (0, 0)),
        ],
        out_specs=pl.BlockSpec((bm, d2), lambda i: (i, 0)),
        out_shape=jax.ShapeDtypeStruct((m, d2), jnp.float32),
    )(a0, a1, h1, deg0, deg1, b1, w2)


def _combine_bias(g0, g1, h2, deg0, deg1, b2):
    m, d = h2.shape
    bm = 1024

    def body(g0_ref, g1_ref, h2_ref, d0_ref, d1_ref, b_ref, o_ref):
        deg = d0_ref[...] + d1_ref[...] + 1.0
        dis = lax.rsqrt(deg)
        o_ref[...] = (g0_ref[...] + g1_ref[...] + h2_ref[...]) * dis + b_ref[...]

    return pl.pallas_call(
        body,
        grid=(m // bm,),
        in_specs=[
            pl.BlockSpec((bm, d), lambda i: (i, 0)),
            pl.BlockSpec((bm, d), lambda i: (i, 0)),
            pl.BlockSpec((bm, d), lambda i: (i, 0)),
            pl.BlockSpec((bm, 1), lambda i: (i, 0)),
            pl.BlockSpec((bm, 1), lambda i: (i, 0)),
            pl.BlockSpec((1, d), lambda i: (0, 0)),
        ],
        out_specs=pl.BlockSpec((bm, d), lambda i: (i, 0)),
        out_shape=jax.ShapeDtypeStruct((m, d), jnp.float32),
    )(g0, g1, h2, deg0, deg1, b2)


def kernel(x, edge_index, W1, b1, W2, b2):
    n, f_in = x.shape
    e = edge_index.shape[1]
    junk = n

    epw = ((e + NW * EDGE_K - 1) // (NW * EDGE_K)) * EDGE_K
    e_pad = epw * NW
    nchunks = epw // EDGE_K
    src = jnp.full((e_pad,), junk, jnp.int32).at[:e].set(
        edge_index[0].astype(jnp.int32)
    )
    dst = jnp.full((e_pad,), junk, jnp.int32).at[:e].set(
        edge_index[1].astype(jnp.int32)
    )
    src = src.reshape(NW, nchunks, EDGE_K)
    dst = dst.reshape(NW, nchunks, EDGE_K)

    xp = jnp.zeros((N_PAD, f_in), x.dtype).at[:n].set(x)

    degp = _deg_kernel(dst, nchunks)
    deg0 = degp[0].reshape(N_PAD, 1)
    deg1 = degp[1].reshape(N_PAD, 1)

    h1 = _mm_scale(xp, W1, deg0, deg1)
    agg1 = _agg_kernel(src, dst, h1, h1.shape[1], nchunks)
    h2 = _combine_relu_mm_scale(
        agg1[0], agg1[1], h1, deg0, deg1, b1.reshape(1, -1), W2
    )
    agg2 = _agg_kernel(src, dst, h2, h2.shape[1], nchunks)
    out = _combine_bias(agg2[0], agg2[1], h2, deg0, deg1, b2.reshape(1, -1))
    return out[:n]


# speedup vs baseline: 6.7922x; 6.7922x over previous
import functools

import jax
import jax.numpy as jnp
from jax import lax
from jax.experimental import pallas as pl
from jax.experimental.pallas import tpu as pltpu
from jax.experimental.pallas import tpu_sc as plsc

N_PAD = 10240
HALF = N_PAD // 2
ACC_R = HALF + 128
NC, NS = 2, 16
EDGE_K = 128
D = 128


def _mesh():
    return plsc.VectorSubcoreMesh(
        core_axis_name="c", subcore_axis_name="s", num_cores=NC, num_subcores=NS
    )


def _localize_dst(dst_v, nchunks, c):
    base = c * HALF

    def body(t, _):
        row = t // (EDGE_K // 16)
        lane = (t % (EDGE_K // 16)) * 16
        v = dst_v[row, pl.ds(lane, 16)]
        local = v - base
        ok = jnp.logical_and(local >= 0, local < HALF)
        dst_v[row, pl.ds(lane, 16)] = jnp.where(ok, local, HALF)
        return 0

    lax.fori_loop(0, nchunks * (EDGE_K // 16), body, 0)


def _zero_acc(acc, zero_v, s):
    rpt = ACC_R // NS
    base = s * rpt
    pltpu.sync_copy(zero_v, acc.at[pl.ds(base, EDGE_K)])
    pltpu.sync_copy(zero_v, acc.at[pl.ds(base + EDGE_K, EDGE_K)])
    pltpu.sync_copy(
        zero_v.at[pl.ds(0, rpt - 2 * EDGE_K)],
        acc.at[pl.ds(base + 2 * EDGE_K, rpt - 2 * EDGE_K)],
    )


def _flush_acc(acc, out_hbm, c, s):
    rpt = ACC_R // NS
    base = s * rpt
    pltpu.sync_copy(acc.at[pl.ds(base, EDGE_K)], out_hbm.at[c, pl.ds(base, EDGE_K)])
    pltpu.sync_copy(
        acc.at[pl.ds(base + EDGE_K, EDGE_K)],
        out_hbm.at[c, pl.ds(base + EDGE_K, EDGE_K)],
    )
    pltpu.sync_copy(
        acc.at[pl.ds(base + 2 * EDGE_K, rpt - 2 * EDGE_K)],
        out_hbm.at[c, pl.ds(base + 2 * EDGE_K, rpt - 2 * EDGE_K)],
    )


def _fill_const(ref, nchunks_16, value):
    def body(t, _):
        ref[t // (D // 16), pl.ds((t % (D // 16)) * 16, 16)] = jnp.full(
            (16,), value, jnp.float32
        )
        return 0

    lax.fori_loop(0, nchunks_16, body, 0)


def _deg_kernel(dst, nchunks):
    @functools.partial(
        pl.kernel,
        out_type=jax.ShapeDtypeStruct((NC, ACC_R, D), jnp.float32),
        mesh=_mesh(),
        scratch_types=[
            pltpu.VMEM((nchunks, EDGE_K), jnp.int32),
            pltpu.VMEM((EDGE_K, D), jnp.float32),
            pltpu.VMEM((EDGE_K, D), jnp.float32),
            pltpu.VMEM_SHARED((ACC_R, D), jnp.float32),
        ],
    )
    def k(dst_hbm, out_hbm, dst_v, ones_v, zero_v, acc):
        c = lax.axis_index("c")
        s = lax.axis_index("s")

        pltpu.sync_copy(dst_hbm.at[s], dst_v)
        _localize_dst(dst_v, nchunks, c)
        _fill_const(ones_v, EDGE_K * (D // 16), 1.0)
        _fill_const(zero_v, EDGE_K * (D // 16), 0.0)
        _zero_acc(acc, zero_v, s)
        plsc.subcore_barrier()

        def body(j, _):
            pltpu.sync_copy(ones_v, acc.at[dst_v.at[j]], add=True)
            return 0

        lax.fori_loop(0, nchunks, body, 0)
        plsc.subcore_barrier()
        _flush_acc(acc, out_hbm, c, s)

    return k(dst)


def _agg_kernel(src, dst, h, nchunks):
    @functools.partial(
        pl.kernel,
        out_type=jax.ShapeDtypeStruct((NC, ACC_R, D), jnp.float32),
        mesh=_mesh(),
        scratch_types=[
            pltpu.VMEM((nchunks, EDGE_K), jnp.int32),
            pltpu.VMEM((nchunks, EDGE_K), jnp.int32),
            pltpu.VMEM((2, EDGE_K, D), jnp.float32),
            pltpu.VMEM((EDGE_K, D), jnp.float32),
            pltpu.VMEM_SHARED((ACC_R, D), jnp.float32),
            pltpu.SemaphoreType.DMA,
            pltpu.SemaphoreType.DMA,
        ],
    )
    def k(src_hbm, dst_hbm, h_hbm, out_hbm, src_v, dst_v, rows_v, zero_v, acc,
          sem0, sem1):
        c = lax.axis_index("c")
        s = lax.axis_index("s")

        pltpu.sync_copy(src_hbm.at[s], src_v)
        pltpu.sync_copy(dst_hbm.at[s], dst_v)
        _localize_dst(dst_v, nchunks, c)
        _fill_const(zero_v, EDGE_K * (D // 16), 0.0)
        _zero_acc(acc, zero_v, s)
        plsc.subcore_barrier()

        pltpu.async_copy(h_hbm.at[src_v.at[0]], rows_v.at[0], sem0)

        def body2(jj, _):
            j0 = jj * 2
            pltpu.make_async_copy(h_hbm.at[src_v.at[j0]], rows_v.at[0], sem0).wait()

            @pl.when(j0 + 1 < nchunks)
            def _():
                pltpu.async_copy(h_hbm.at[src_v.at[j0 + 1]], rows_v.at[1], sem1)

            pltpu.sync_copy(rows_v.at[0], acc.at[dst_v.at[j0]], add=True)

            @pl.when(j0 + 1 < nchunks)
            def _():
                pltpu.make_async_copy(
                    h_hbm.at[src_v.at[j0 + 1]], rows_v.at[1], sem1
                ).wait()

                @pl.when(j0 + 2 < nchunks)
                def _():
                    pltpu.async_copy(h_hbm.at[src_v.at[j0 + 2]], rows_v.at[0], sem0)

                pltpu.sync_copy(rows_v.at[1], acc.at[dst_v.at[j0 + 1]], add=True)

            return 0

        lax.fori_loop(0, (nchunks + 1) // 2, body2, 0)
        plsc.subcore_barrier()
        _flush_acc(acc, out_hbm, c, s)

    return k(src, dst, h)


_BM = 1024
_NB5 = HALF // _BM


def _mm_scale(x, w, deg):
    m, kdim = x.shape
    _, d = w.shape

    def body(x_ref, w_ref, dg_ref, o_ref):
        dis = lax.rsqrt(dg_ref[...])
        o_ref[...] = (
            jnp.dot(x_ref[...], w_ref[...], preferred_element_type=jnp.float32) * dis
        )

    return pl.pallas_call(
        body,
        grid=(m // _BM,),
        in_specs=[
            pl.BlockSpec((_BM, kdim), lambda i: (i, 0)),
            pl.BlockSpec((kdim, d), lambda i: (0, 0)),
            pl.BlockSpec((_BM, 1), lambda i: (i, 0)),
        ],
        out_specs=pl.BlockSpec((_BM, d), lambda i: (i, 0)),
        out_shape=jax.ShapeDtypeStruct((m, d), jnp.float32),
    )(x, w, deg)


def _combine_relu_mm_scale(p, h1, deg, b1, w2):
    m, d = h1.shape
    _, d2 = w2.shape

    def body(p_ref, h1_ref, dg_ref, b_ref, w_ref, o_ref):
        dis = lax.rsqrt(dg_ref[...])
        z = jnp.maximum((p_ref[0] + h1_ref[...]) * dis + b_ref[...], 0.0)
        r = jnp.dot(z, w_ref[...], preferred_element_type=jnp.float32) * dis
        o_ref[...] = jnp.concatenate(
            [r, jnp.zeros((_BM, d - d2), jnp.float32)], axis=1
        )

    return pl.pallas_call(
        body,
        grid=(m // _BM,),
        in_specs=[
            pl.BlockSpec((1, _BM, d), lambda i: (i // _NB5, i % _NB5, 0)),
            pl.BlockSpec((_BM, d), lambda i: (i, 0)),
            pl.BlockSpec((_BM, 1), lambda i: (i, 0)),
            pl.BlockSpec((1, d), lambda i: (0, 0)),
            pl.BlockSpec((d, d2), lambda i: (0, 0)),
        ],
        out_specs=pl.BlockSpec((_BM, d), lambda i: (i, 0)),
        out_shape=jax.ShapeDtypeStruct((m, d), jnp.float32),
    )(p, h1, deg, b1, w2)


def _combine_bias(p, h2, deg, b2, d2):
    m, d = h2.shape

    def body(p_ref, h2_ref, dg_ref, b_ref, o_ref):
        dis = lax.rsqrt(dg_ref[...])
        o_ref[...] = (p_ref[0][:, :d2] + h2_ref[:, :d2]) * dis + b_ref[...]

    return pl.pallas_call(
        body,
        grid=(m // _BM,),
        in_specs=[
            pl.BlockSpec((1, _BM, d), lambda i: (i // _NB5, i % _NB5, 0)),
            pl.BlockSpec((_BM, d), lambda i: (i, 0)),
            pl.BlockSpec((_BM, 1), lambda i: (i, 0)),
            pl.BlockSpec((1, d2), lambda i: (0, 0)),
        ],
        out_specs=pl.BlockSpec((_BM, d2), lambda i: (i, 0)),
        out_shape=jax.ShapeDtypeStruct((m, d2), jnp.float32),
    )(p, h2, deg, b2)


def kernel(x, edge_index, W1, b1, W2, b2):
    n, f_in = x.shape
    e = edge_index.shape[1]
    junk = n

    eps = ((e + NS * EDGE_K - 1) // (NS * EDGE_K)) * EDGE_K
    e_pad = eps * NS
    nchunks = eps // EDGE_K
    src = jnp.full((e_pad,), junk, jnp.int32).at[:e].set(
        edge_index[0].astype(jnp.int32)
    )
    dst = jnp.full((e_pad,), junk, jnp.int32).at[:e].set(
        edge_index[1].astype(jnp.int32)
    )
    src = src.reshape(NS, nchunks, EDGE_K)
    dst = dst.reshape(NS, nchunks, EDGE_K)

    xp = jnp.zeros((N_PAD, f_in), x.dtype).at[:n].set(x)

    degp = _deg_kernel(dst, nchunks)
    deg = (
        jnp.concatenate([degp[0, :HALF, 0], degp[1, :HALF, 0]]) + 1.0
    ).reshape(N_PAD, 1)

    h1 = _mm_scale(xp, W1, deg)
    agg1 = _agg_kernel(src, dst, h1, nchunks)
    h2 = _combine_relu_mm_scale(agg1, h1, deg, b1.reshape(1, -1), W2)
    agg2 = _agg_kernel(src, dst, h2, nchunks)
    out = _combine_bias(agg2, h2, deg, b2.reshape(1, -1), W2.shape[1])
    return out[:n]


# speedup vs baseline: 7.7842x; 1.1460x over previous
import functools

import jax
import jax.numpy as jnp
from jax import lax
from jax.experimental import pallas as pl
from jax.experimental.pallas import tpu as pltpu
from jax.experimental.pallas import tpu_sc as plsc

N_PAD = 10240
HALF = N_PAD // 2
ACC_R = HALF + 128
NC, NS = 2, 16
EDGE_K = 128
D = 128
NBUF = 4


def _mesh():
    return plsc.VectorSubcoreMesh(
        core_axis_name="c", subcore_axis_name="s", num_cores=NC, num_subcores=NS
    )


def _localize_dst(dst_v, nchunks, c):
    base = c * HALF

    def body(t, _):
        row = t // (EDGE_K // 16)
        lane = (t % (EDGE_K // 16)) * 16
        v = dst_v[row, pl.ds(lane, 16)]
        local = v - base
        ok = jnp.logical_and(local >= 0, local < HALF)
        dst_v[row, pl.ds(lane, 16)] = jnp.where(
            ok, local, HALF + jnp.bitwise_and(v, 127)
        )
        return 0

    lax.fori_loop(0, nchunks * (EDGE_K // 16), body, 0)


def _zero_acc(acc, zero_v, s):
    rpt = ACC_R // NS
    base = s * rpt
    pltpu.sync_copy(zero_v, acc.at[pl.ds(base, EDGE_K)])
    pltpu.sync_copy(zero_v, acc.at[pl.ds(base + EDGE_K, EDGE_K)])
    pltpu.sync_copy(
        zero_v.at[pl.ds(0, rpt - 2 * EDGE_K)],
        acc.at[pl.ds(base + 2 * EDGE_K, rpt - 2 * EDGE_K)],
    )


def _flush_acc(acc, out_hbm, c, s):
    rpt = ACC_R // NS
    base = s * rpt
    pltpu.sync_copy(acc.at[pl.ds(base, EDGE_K)], out_hbm.at[c, pl.ds(base, EDGE_K)])
    pltpu.sync_copy(
        acc.at[pl.ds(base + EDGE_K, EDGE_K)],
        out_hbm.at[c, pl.ds(base + EDGE_K, EDGE_K)],
    )
    pltpu.sync_copy(
        acc.at[pl.ds(base + 2 * EDGE_K, rpt - 2 * EDGE_K)],
        out_hbm.at[c, pl.ds(base + 2 * EDGE_K, rpt - 2 * EDGE_K)],
    )


def _fill_zero2d(ref, n16):
    def body(t, _):
        ref[t // (D // 16), pl.ds((t % (D // 16)) * 16, 16)] = jnp.zeros(
            (16,), jnp.float32
        )
        return 0

    lax.fori_loop(0, n16, body, 0)


def _deg_kernel(dst, nchunks):
    @functools.partial(
        pl.kernel,
        out_type=jax.ShapeDtypeStruct((NC, ACC_R, D), jnp.float32),
        mesh=_mesh(),
        scratch_types=[
            pltpu.VMEM((nchunks, EDGE_K), jnp.int32),
            pltpu.VMEM((EDGE_K, D), jnp.float32),
            pltpu.VMEM_SHARED((ACC_R, D), jnp.float32),
        ]
        + [pltpu.SemaphoreType.DMA] * NBUF,
    )
    def k(dst_hbm, out_hbm, dst_v, ones_v, acc, *sems):
        c = lax.axis_index("c")
        s = lax.axis_index("s")

        pltpu.sync_copy(dst_hbm.at[s], dst_v)
        _localize_dst(dst_v, nchunks, c)
        _fill_zero2d(ones_v, EDGE_K * (D // 16))
        _zero_acc(acc, ones_v, s)

        ones16 = jnp.ones((16,), jnp.float32)

        def obody(t, _):
            ones_v[t // (D // 16), pl.ds((t % (D // 16)) * 16, 16)] = ones16
            return 0

        lax.fori_loop(0, EDGE_K * (D // 16), obody, 0)
        plsc.subcore_barrier()

        ngroups = (nchunks + NBUF - 1) // NBUF

        def group(jj, _):
            t0 = jj * NBUF
            for b in range(NBUF):
                t = t0 + b

                @pl.when(t < nchunks)
                def _():
                    @pl.when(t >= NBUF)
                    def _():
                        pltpu.make_async_copy(
                            ones_v, acc.at[dst_v.at[t - NBUF]], sems[b]
                        ).wait()

                    pltpu.async_copy(ones_v, acc.at[dst_v.at[t]], sems[b], add=True)

            return 0

        lax.fori_loop(0, ngroups, group, 0)

        for t in range(max(0, nchunks - NBUF), nchunks):
            pltpu.make_async_copy(ones_v, acc.at[dst_v.at[t]], sems[t % NBUF]).wait()
        plsc.subcore_barrier()
        _flush_acc(acc, out_hbm, c, s)

    return k(dst)


def _agg_kernel(src, dst, h, nchunks):
    @functools.partial(
        pl.kernel,
        out_type=jax.ShapeDtypeStruct((NC, ACC_R, D), jnp.float32),
        mesh=_mesh(),
        scratch_types=[
            pltpu.VMEM((nchunks, EDGE_K), jnp.int32),
            pltpu.VMEM((nchunks, EDGE_K), jnp.int32),
            pltpu.VMEM((NBUF, EDGE_K, D), jnp.float32),
            pltpu.VMEM_SHARED((ACC_R, D), jnp.float32),
        ]
        + [pltpu.SemaphoreType.DMA] * (2 * NBUF),
    )
    def k(src_hbm, dst_hbm, h_hbm, out_hbm, src_v, dst_v, rows_v, acc,
          *sems):
        gsem = sems[:NBUF]
        ssem = sems[NBUF:]
        c = lax.axis_index("c")
        s = lax.axis_index("s")

        pltpu.sync_copy(src_hbm.at[s], src_v)
        pltpu.sync_copy(dst_hbm.at[s], dst_v)
        _localize_dst(dst_v, nchunks, c)
        _fill_zero2d(rows_v.at[0], EDGE_K * (D // 16))
        _zero_acc(acc, rows_v.at[0], s)
        plsc.subcore_barrier()

        def gather(t, b):
            pltpu.async_copy(h_hbm.at[src_v.at[t]], rows_v.at[b], gsem[b])

        def scatter(t, b):
            pltpu.async_copy(rows_v.at[b], acc.at[dst_v.at[t]], ssem[b], add=True)

        gather(0, 0)
        if nchunks > 1:
            gather(1, 1)

        ngroups = (nchunks + NBUF - 1) // NBUF

        def group(jj, _):
            t0 = jj * NBUF
            for b in range(NBUF):
                t = t0 + b

                @pl.when(t < nchunks)
                def _():
                    pltpu.make_async_copy(
                        h_hbm.at[src_v.at[t]], rows_v.at[b], gsem[b]
                    ).wait()
                    b2 = (b + 2) % NBUF

                    @pl.when(t >= 2)
                    def _():
                        pltpu.make_async_copy(
                            rows_v.at[b2], acc.at[dst_v.at[t - 2]], ssem[b2]
                        ).wait()

                    @pl.when(t + 2 < nchunks)
                    def _():
                        gather(t + 2, b2)

                    scatter(t, b)

            return 0

        lax.fori_loop(0, ngroups, group, 0)

        for t in (nchunks - 2, nchunks - 1):
            if t >= 0:
                b = t % NBUF
                pltpu.make_async_copy(
                    rows_v.at[b], acc.at[dst_v.at[t]], ssem[b]
                ).wait()
        plsc.subcore_barrier()
        _flush_acc(acc, out_hbm, c, s)

    return k(src, dst, h)


_BM = 1024
_NB5 = HALF // _BM


def _mm_scale(x, w, deg):
    m, kdim = x.shape
    _, d = w.shape

    def body(x_ref, w_ref, d_ref, o_ref):
        dis = lax.rsqrt(d_ref[...] + 1.0)
        o_ref[...] = (
            jnp.dot(x_ref[...], w_ref[...], preferred_element_type=jnp.float32) * dis
        )

    return pl.pallas_call(
        body,
        grid=(m // _BM,),
        in_specs=[
            pl.BlockSpec((_BM, kdim), lambda i: (i, 0)),
            pl.BlockSpec((kdim, d), lambda i: (0, 0)),
            pl.BlockSpec((_BM, 1), lambda i: (i, 0)),
        ],
        out_specs=pl.BlockSpec((_BM, d), lambda i: (i, 0)),
        out_shape=jax.ShapeDtypeStruct((m, d), jnp.float32),
    )(x, w, deg)


def _combine_relu_mm_scale(p, h1, deg, b1, w2):
    m, d = h1.shape
    _, d2 = w2.shape

    def body(p_ref, h1_ref, d_ref, b_ref, w_ref, o_ref):
        dis = lax.rsqrt(d_ref[...] + 1.0)
        z = jnp.maximum((p_ref[0] + h1_ref[...]) * dis + b_ref[...], 0.0)
        r = jnp.dot(z, w_ref[...], preferred_element_type=jnp.float32) * dis
        o_ref[...] = jnp.concatenate(
            [r, jnp.zeros((_BM, d - d2), jnp.float32)], axis=1
        )

    return pl.pallas_call(
        body,
        grid=(m // _BM,),
        in_specs=[
            pl.BlockSpec((1, _BM, d), lambda i: (i // _NB5, i % _NB5, 0)),
            pl.BlockSpec((_BM, d), lambda i: (i, 0)),
            pl.BlockSpec((_BM, 1), lambda i: (i, 0)),
            pl.BlockSpec((1, d), lambda i: (0, 0)),
            pl.BlockSpec((d, d2), lambda i: (0, 0)),
        ],
        out_specs=pl.BlockSpec((_BM, d), lambda i: (i, 0)),
        out_shape=jax.ShapeDtypeStruct((m, d), jnp.float32),
    )(p, h1, deg, b1, w2)


def _combine_bias(p, h2, deg, b2, d2):
    m, d = h2.shape

    def body(p_ref, h2_ref, d_ref, b_ref, o_ref):
        dis = lax.rsqrt(d_ref[...] + 1.0)
        o_ref[...] = (p_ref[0][:, :d2] + h2_ref[:, :d2]) * dis + b_ref[...]

    return pl.pallas_call(
        body,
        grid=(m // _BM,),
        in_specs=[
            pl.BlockSpec((1, _BM, d), lambda i: (i // _NB5, i % _NB5, 0)),
            pl.BlockSpec((_BM, d), lambda i: (i, 0)),
            pl.BlockSpec((_BM, 1), lambda i: (i, 0)),
            pl.BlockSpec((1, d2), lambda i: (0, 0)),
        ],
        out_specs=pl.BlockSpec((_BM, d2), lambda i: (i, 0)),
        out_shape=jax.ShapeDtypeStruct((m, d2), jnp.float32),
    )(p, h2, deg, b2)


def kernel(x, edge_index, W1, b1, W2, b2):
    n, f_in = x.shape
    e = edge_index.shape[1]
    junk = n

    eps = ((e + NS * EDGE_K - 1) // (NS * EDGE_K)) * EDGE_K
    e_pad = eps * NS
    nchunks = eps // EDGE_K
    src = jnp.full((e_pad,), junk, jnp.int32).at[:e].set(
        edge_index[0].astype(jnp.int32)
    )
    dst = jnp.full((e_pad,), junk, jnp.int32).at[:e].set(
        edge_index[1].astype(jnp.int32)
    )
    src = src.reshape(NS, nchunks, EDGE_K)
    dst = dst.reshape(NS, nchunks, EDGE_K)

    xp = jnp.zeros((N_PAD, f_in), x.dtype).at[:n].set(x)

    degp = _deg_kernel(dst, nchunks)
    deg = jnp.concatenate(
        [degp[0, :HALF, 0], degp[1, :HALF, 0]]
    ).reshape(N_PAD, 1)

    h1 = _mm_scale(xp, W1, deg)
    agg1 = _agg_kernel(src, dst, h1, nchunks)
    h2 = _combine_relu_mm_scale(agg1, h1, deg, b1.reshape(1, -1), W2)
    agg2 = _agg_kernel(src, dst, h2, nchunks)
    out = _combine_bias(agg2, h2, deg, b2.reshape(1, -1), W2.shape[1])
    return out[:n]
